# baseline (speedup 1.0000x reference)
import jax
import jax.numpy as jnp
from jax.experimental import pallas as pl
from jax.experimental.pallas import tpu as pltpu

B, T, D, H = 16, 4096, 128, 128
N_BLK = 8
T_BLK = T // N_BLK


def _rnn_kernel(xt_ref, h0_ref, wih_ref, whh_ref, wout_ref, b_ref, bout_ref,
                out_ref, h_ref, az_ref):
    i = pl.program_id(0)

    @pl.when(i == 0)
    def _():
        h_ref[:] = h0_ref[:]

    xb = xt_ref[:].reshape(T_BLK * B, D)
    a = jnp.dot(xb, wih_ref[:], preferred_element_type=jnp.float32)
    az_ref[:] = (a + b_ref[:]).astype(jnp.bfloat16).reshape(T_BLK, B, H)

    whh = whh_ref[:]

    def step(t, h):
        hn = jnp.tanh(az_ref[t].astype(jnp.float32)
                      + jnp.dot(h, whh, preferred_element_type=jnp.float32))
        hb = hn.astype(jnp.bfloat16)
        az_ref[t] = hb
        return hb

    h_last = jax.lax.fori_loop(0, T_BLK, step, h_ref[:], unroll=8)
    h_ref[:] = h_last

    z = az_ref[:].reshape(T_BLK * B, H)
    out_ref[:] = (jnp.dot(z, wout_ref[:], preferred_element_type=jnp.float32)
                  + bout_ref[:]).reshape(T_BLK, B, D)


def kernel(x, h0, W_ih, W_hh, b_ih, b_hh, W_out, b_out):
    xt = jnp.transpose(x, (1, 0, 2)).astype(jnp.bfloat16)
    b = (b_ih + b_hh).reshape(1, H)
    bo = b_out.reshape(1, D)
    out_t = pl.pallas_call(
        _rnn_kernel,
        grid=(N_BLK,),
        in_specs=[
            pl.BlockSpec((T_BLK, B, D), lambda i: (i, 0, 0)),
            pl.BlockSpec((B, H), lambda i: (0, 0)),
            pl.BlockSpec((D, H), lambda i: (0, 0)),
            pl.BlockSpec((H, H), lambda i: (0, 0)),
            pl.BlockSpec((H, D), lambda i: (0, 0)),
            pl.BlockSpec((1, H), lambda i: (0, 0)),
            pl.BlockSpec((1, D), lambda i: (0, 0)),
        ],
        out_specs=pl.BlockSpec((T_BLK, B, D), lambda i: (i, 0, 0)),
        out_shape=jax.ShapeDtypeStruct((T, B, D), jnp.float32),
        scratch_shapes=[
            pltpu.VMEM((B, H), jnp.bfloat16),
            pltpu.VMEM((T_BLK, B, H), jnp.bfloat16),
        ],
    )(xt, h0.astype(jnp.bfloat16), W_ih.T.astype(jnp.bfloat16),
      W_hh.T.astype(jnp.bfloat16), W_out.T.astype(jnp.bfloat16), b, bo)
    return jnp.transpose(out_t, (1, 0, 2))


# speedup vs baseline: 1.0427x; 1.0427x over previous
import jax
import jax.numpy as jnp
from jax.experimental import pallas as pl
from jax.experimental.pallas import tpu as pltpu

B, T, D, H = 16, 4096, 128, 128
N_BLK = 8
T_BLK = T // N_BLK


def _rnn_kernel(x_ref, h0_ref, wih_ref, whh_ref, wout_ref, b_ref, bout_ref,
                out_ref, h_ref, az_ref):
    i = pl.program_id(0)

    @pl.when(i == 0)
    def _():
        h_ref[:] = h0_ref[:]

    xb = x_ref[:].reshape(B * T_BLK, D)
    a = jnp.dot(xb, wih_ref[:], preferred_element_type=jnp.float32)
    az_ref[:] = (a + b_ref[:]).reshape(B, T_BLK, H)

    whh = whh_ref[:]

    def step(t, h):
        a_t = az_ref[:, pl.ds(t, 1), :].reshape(B, H)
        hn = jnp.tanh(a_t + jnp.dot(h, whh,
                                    preferred_element_type=jnp.float32))
        az_ref[:, pl.ds(t, 1), :] = hn.reshape(B, 1, H)
        return hn.astype(jnp.bfloat16)

    h_last = jax.lax.fori_loop(0, T_BLK, step,
                               h_ref[:].astype(jnp.bfloat16), unroll=8)
    h_ref[:] = h_last.astype(jnp.float32)

    z = az_ref[:].reshape(B * T_BLK, H)
    out_ref[:] = (jnp.dot(z, wout_ref[:], preferred_element_type=jnp.float32)
                  + bout_ref[:]).reshape(B, T_BLK, D)


def kernel(x, h0, W_ih, W_hh, b_ih, b_hh, W_out, b_out):
    b = (b_ih + b_hh).reshape(1, H)
    bo = b_out.reshape(1, D)
    out = pl.pallas_call(
        _rnn_kernel,
        grid=(N_BLK,),
        in_specs=[
            pl.BlockSpec((B, T_BLK, D), lambda i: (0, i, 0)),
            pl.BlockSpec((B, H), lambda i: (0, 0)),
            pl.BlockSpec((D, H), lambda i: (0, 0)),
            pl.BlockSpec((H, H), lambda i: (0, 0)),
            pl.BlockSpec((H, D), lambda i: (0, 0)),
            pl.BlockSpec((1, H), lambda i: (0, 0)),
            pl.BlockSpec((1, D), lambda i: (0, 0)),
        ],
        out_specs=pl.BlockSpec((B, T_BLK, D), lambda i: (0, i, 0)),
        out_shape=jax.ShapeDtypeStruct((B, T, D), jnp.float32),
        scratch_shapes=[
            pltpu.VMEM((B, H), jnp.float32),
            pltpu.VMEM((B, T_BLK, H), jnp.float32),
        ],
    )(x, h0, W_ih.T, W_hh.T.astype(jnp.bfloat16), W_out.T, b, bo)
    return out


# speedup vs baseline: 1.1196x; 1.0737x over previous
import jax
import jax.numpy as jnp
from jax.experimental import pallas as pl
from jax.experimental.pallas import tpu as pltpu

B, T, D, H = 16, 4096, 128, 128
N_BLK = 8
T_BLK = T // N_BLK


def _rnn_kernel(x_ref, h0_ref, wih_ref, whh_ref, wout_ref, b_ref, bout_ref,
                out_ref, h_ref, az_ref, zs_ref, at_ref):
    i = pl.program_id(0)

    @pl.when(i == 0)
    def _():
        h_ref[:] = h0_ref[:]

    xb = x_ref[:].reshape(B * T_BLK, D)
    a = jnp.dot(xb, wih_ref[:], preferred_element_type=jnp.float32)
    az_ref[:] = (a + b_ref[:]).reshape(B, T_BLK, H)

    whh = whh_ref[:]

    def step(t, h):
        at_ref[:] = az_ref[:, pl.ds(t, 1), :].reshape(B, H)
        hn = jnp.tanh(at_ref[:] + jnp.dot(h, whh,
                                          preferred_element_type=jnp.float32))
        zs_ref[:, pl.ds(t, 1), :] = hn.reshape(B, 1, H)
        return hn.astype(jnp.bfloat16)

    h_last = jax.lax.fori_loop(0, T_BLK, step,
                               h_ref[:].astype(jnp.bfloat16), unroll=8)
    h_ref[:] = h_last.astype(jnp.float32)

    z = zs_ref[:].reshape(B * T_BLK, H)
    out_ref[:] = (jnp.dot(z, wout_ref[:], preferred_element_type=jnp.float32)
                  + bout_ref[:]).reshape(B, T_BLK, D)


def kernel(x, h0, W_ih, W_hh, b_ih, b_hh, W_out, b_out):
    b = (b_ih + b_hh).reshape(1, H)
    bo = b_out.reshape(1, D)
    out = pl.pallas_call(
        _rnn_kernel,
        grid=(N_BLK,),
        in_specs=[
            pl.BlockSpec((B, T_BLK, D), lambda i: (0, i, 0)),
            pl.BlockSpec((B, H), lambda i: (0, 0)),
            pl.BlockSpec((D, H), lambda i: (0, 0)),
            pl.BlockSpec((H, H), lambda i: (0, 0)),
            pl.BlockSpec((H, D), lambda i: (0, 0)),
            pl.BlockSpec((1, H), lambda i: (0, 0)),
            pl.BlockSpec((1, D), lambda i: (0, 0)),
        ],
        out_specs=pl.BlockSpec((B, T_BLK, D), lambda i: (0, i, 0)),
        out_shape=jax.ShapeDtypeStruct((B, T, D), jnp.float32),
        scratch_shapes=[
            pltpu.VMEM((B, H), jnp.float32),
            pltpu.VMEM((B, T_BLK, H), jnp.float32),
            pltpu.VMEM((B, T_BLK, H), jnp.float32),
            pltpu.VMEM((B, H), jnp.float32),
        ],
    )(x, h0, W_ih.T, W_hh.T.astype(jnp.bfloat16), W_out.T, b, bo)
    return out


# speedup vs baseline: 1.1319x; 1.0110x over previous
import jax
import jax.numpy as jnp
from jax.experimental import pallas as pl
from jax.experimental.pallas import tpu as pltpu

B, T, D, H = 16, 4096, 128, 128
N_BLK = 4
T_BLK = T // N_BLK


def _rnn_kernel(x_ref, h0_ref, wih_ref, whh_ref, wout_ref, b_ref, bout_ref,
                out_ref, h_ref, az_ref, zs_ref, at_ref):
    i = pl.program_id(0)

    @pl.when(i == 0)
    def _():
        h_ref[:] = h0_ref[:]

    xb = x_ref[:].reshape(B * T_BLK, D)
    a = jnp.dot(xb, wih_ref[:], preferred_element_type=jnp.float32)
    az_ref[:] = (a + b_ref[:]).reshape(B, T_BLK, H)

    whh = whh_ref[:]

    def step(t, h):
        at_ref[:] = az_ref[:, pl.ds(t, 1), :].reshape(B, H)
        hn = jnp.tanh(at_ref[:] + jnp.dot(h, whh,
                                          preferred_element_type=jnp.float32))
        zs_ref[:, pl.ds(t, 1), :] = hn.reshape(B, 1, H)
        return hn.astype(jnp.bfloat16)

    h_last = jax.lax.fori_loop(0, T_BLK, step,
                               h_ref[:].astype(jnp.bfloat16), unroll=16)
    h_ref[:] = h_last.astype(jnp.float32)

    z = zs_ref[:].reshape(B * T_BLK, H)
    out_ref[:] = (jnp.dot(z, wout_ref[:], preferred_element_type=jnp.float32)
                  + bout_ref[:]).reshape(B, T_BLK, D)


def kernel(x, h0, W_ih, W_hh, b_ih, b_hh, W_out, b_out):
    b = (b_ih + b_hh).reshape(1, H)
    bo = b_out.reshape(1, D)
    out = pl.pallas_call(
        _rnn_kernel,
        grid=(N_BLK,),
        in_specs=[
            pl.BlockSpec((B, T_BLK, D), lambda i: (0, i, 0)),
            pl.BlockSpec((B, H), lambda i: (0, 0)),
            pl.BlockSpec((D, H), lambda i: (0, 0)),
            pl.BlockSpec((H, H), lambda i: (0, 0)),
            pl.BlockSpec((H, D), lambda i: (0, 0)),
            pl.BlockSpec((1, H), lambda i: (0, 0)),
            pl.BlockSpec((1, D), lambda i: (0, 0)),
        ],
        out_specs=pl.BlockSpec((B, T_BLK, D), lambda i: (0, i, 0)),
        out_shape=jax.ShapeDtypeStruct((B, T, D), jnp.float32),
        scratch_shapes=[
            pltpu.VMEM((B, H), jnp.float32),
            pltpu.VMEM((B, T_BLK, H), jnp.float32),
            pltpu.VMEM((B, T_BLK, H), jnp.float32),
            pltpu.VMEM((B, H), jnp.float32),
        ],
    )(x, h0, W_ih.T, W_hh.T.astype(jnp.bfloat16), W_out.T, b, bo)
    return out
